# Initial kernel scaffold; baseline (speedup 1.0000x reference)
import jax
import jax.numpy as jnp
from jax.experimental import pallas as pl


def kernel(emb, center_words, center_words_len, mwe_words):
    raise NotImplementedError("write your pallas kernel here")


# speedup vs baseline: 2.3150x; 2.3150x over previous
import functools

import jax
import jax.numpy as jnp
from jax import lax
from jax.experimental import pallas as pl
from jax.experimental.pallas import tpu as pltpu
from jax.experimental.pallas import tpu_sc as plsc


def _build_sc_kernel(V, D, B, L):
    info = plsc.get_sparse_core_info()
    NC, NS, LN = info.num_cores, info.num_subcores, info.num_lanes
    NW = NC * NS
    assert B % NW == 0 and D % LN == 0
    b_per_w = B // NW
    CB = 16
    n_chunks = b_per_w // CB
    rows_per_chunk = CB * L
    n_vregs = D // LN

    mesh = plsc.VectorSubcoreMesh(core_axis_name="c", subcore_axis_name="s")

    @functools.partial(
        pl.kernel,
        mesh=mesh,
        out_type=jax.ShapeDtypeStruct((NW, LN), jnp.float32),
        scratch_types=[
            pltpu.VMEM((b_per_w * L,), jnp.int32),
            pltpu.VMEM((b_per_w,), jnp.int32),
            pltpu.VMEM((b_per_w,), jnp.int32),
            pltpu.VMEM((rows_per_chunk, D), jnp.float32),
            pltpu.VMEM((CB, D), jnp.float32),
            pltpu.VMEM((LN,), jnp.float32),
            pltpu.SemaphoreType.DMA,
        ],
    )
    def sc_kernel(emb_hbm, cw_hbm, len_hbm, mwe_hbm, out_hbm,
                  idx_v, len_v, midx_v, rows_v, tgt_v, out_v, sem):
        wid = lax.axis_index("s") * NC + lax.axis_index("c")
        base_b = wid * b_per_w

        pltpu.sync_copy(cw_hbm.at[pl.ds(base_b * L, b_per_w * L)], idx_v)
        pltpu.sync_copy(len_hbm.at[pl.ds(base_b, b_per_w)], len_v)
        pltpu.sync_copy(mwe_hbm.at[pl.ds(base_b, b_per_w)], midx_v)

        zeros = jnp.zeros((LN,), jnp.float32)
        sq = (zeros,) * n_vregs

        for cb in range(n_chunks):
            off = cb * rows_per_chunk
            cps = []
            r0 = 0
            while r0 < rows_per_chunk:
                n = min(128, rows_per_chunk - r0)
                cps.append(pltpu.async_copy(
                    emb_hbm.at[idx_v.at[pl.ds(off + r0, n)]],
                    rows_v.at[pl.ds(r0, n)], sem))
                r0 += n
            cps.append(pltpu.async_copy(
                emb_hbm.at[midx_v.at[pl.ds(cb * CB, CB)]], tgt_v, sem))
            for cp in cps:
                cp.wait()

            lens16 = len_v[pl.ds(cb * CB, CB)]

            def b_body(b, sq, _lens16=lens16):
                dnums = lax.GatherDimensionNumbers(
                    offset_dims=(), collapsed_slice_dims=(0,),
                    start_index_map=(0,))
                ln_b = lax.gather(
                    _lens16, jnp.full((LN, 1), b, jnp.int32), dnums,
                    slice_sizes=(1,),
                    mode=lax.GatherScatterMode.PROMISE_IN_BOUNDS)

                inv = jnp.full((LN,), 1.0, jnp.float32) / ln_b.astype(
                    jnp.float32)

                def l_body(l, accs, _b=b):
                    w = jnp.minimum(jnp.maximum(ln_b - l, 0),
                                    1).astype(jnp.float32) * inv
                    r = _b * L + l
                    return tuple(
                        accs[j] + rows_v[r, pl.ds(j * LN, LN)] * w
                        for j in range(n_vregs))

                accs = lax.fori_loop(0, L, l_body, (zeros,) * n_vregs)
                out = []
                for j in range(n_vregs):
                    diff = accs[j] - tgt_v[b, pl.ds(j * LN, LN)]
                    out.append(sq[j] + diff * diff)
                return tuple(out)

            sq = lax.fori_loop(0, CB, b_body, sq)

        total = sq[0]
        for j in range(1, n_vregs):
            total = total + sq[j]
        out_v[...] = total
        pltpu.sync_copy(out_v, out_hbm.at[wid])

    return sc_kernel


def kernel(emb, center_words, center_words_len, mwe_words):
    V, D = emb.shape
    B, L = center_words.shape
    sc = _build_sc_kernel(V, D, B, L)
    partials = sc(emb, center_words.reshape(-1), center_words_len, mwe_words)
    return jnp.sum(partials) / jnp.float32(B * D)


# speedup vs baseline: 2.4507x; 1.0586x over previous
import functools

import jax
import jax.numpy as jnp
from jax import lax
from jax.experimental import pallas as pl
from jax.experimental.pallas import tpu as pltpu
from jax.experimental.pallas import tpu_sc as plsc


def _build_sc_kernel(V, D, B, L):
    info = plsc.get_sparse_core_info()
    NC, NS, LN = info.num_cores, info.num_subcores, info.num_lanes
    NW = NC * NS
    assert B % NW == 0 and D % LN == 0
    b_per_w = B // NW
    CB = 16
    n_chunks = b_per_w // CB
    rows_per_chunk = CB * L
    n_vregs = D // LN

    mesh = plsc.VectorSubcoreMesh(core_axis_name="c", subcore_axis_name="s")

    @functools.partial(
        pl.kernel,
        mesh=mesh,
        out_type=jax.ShapeDtypeStruct((NW, LN), jnp.float32),
        scratch_types=[
            pltpu.VMEM((b_per_w * L,), jnp.int32),
            pltpu.VMEM((b_per_w + LN,), jnp.int32),
            pltpu.VMEM((b_per_w,), jnp.int32),
            pltpu.VMEM((rows_per_chunk, D), jnp.float32),
            pltpu.VMEM((CB, D), jnp.float32),
            pltpu.VMEM((LN,), jnp.float32),
            pltpu.SemaphoreType.DMA,
        ],
    )
    def sc_kernel(emb_hbm, cw_hbm, len_hbm, mwe_hbm, out_hbm,
                  idx_v, len_v, midx_v, rows_v, tgt_v, out_v, sem):
        wid = lax.axis_index("s") * NC + lax.axis_index("c")
        base_b = wid * b_per_w

        pltpu.sync_copy(cw_hbm.at[pl.ds(base_b * L, b_per_w * L)], idx_v)
        pltpu.sync_copy(len_hbm.at[pl.ds(base_b, b_per_w)],
                        len_v.at[pl.ds(0, b_per_w)])
        pltpu.sync_copy(mwe_hbm.at[pl.ds(base_b, b_per_w)], midx_v)

        zeros = jnp.zeros((LN,), jnp.float32)
        sq = (zeros,) * n_vregs

        for cb in range(n_chunks):
            off = cb * rows_per_chunk
            cps = []
            r0 = 0
            while r0 < rows_per_chunk:
                n = min(128, rows_per_chunk - r0)
                cps.append(pltpu.async_copy(
                    emb_hbm.at[idx_v.at[pl.ds(off + r0, n)]],
                    rows_v.at[pl.ds(r0, n)], sem))
                r0 += n
            cps.append(pltpu.async_copy(
                emb_hbm.at[midx_v.at[pl.ds(cb * CB, CB)]], tgt_v, sem))
            for cp in cps:
                cp.wait()

            def b_body(b, sq, _cb=cb):
                ln = len_v[pl.ds(_cb * CB + b, LN)][0]
                inv = jnp.full((LN,), 1.0, jnp.float32) / jnp.full(
                    (LN,), ln).astype(jnp.float32)

                def l_body(l, accs, _b=b):
                    r = _b * L + l
                    return tuple(accs[j] + rows_v[r, pl.ds(j * LN, LN)]
                                 for j in range(n_vregs))

                accs = lax.fori_loop(0, ln, l_body, (zeros,) * n_vregs)
                out = []
                for j in range(n_vregs):
                    diff = accs[j] * inv - tgt_v[b, pl.ds(j * LN, LN)]
                    out.append(sq[j] + diff * diff)
                return tuple(out)

            sq = lax.fori_loop(0, CB, b_body, sq)

        total = sq[0]
        for j in range(1, n_vregs):
            total = total + sq[j]
        out_v[...] = total
        pltpu.sync_copy(out_v, out_hbm.at[wid])

    return sc_kernel


def kernel(emb, center_words, center_words_len, mwe_words):
    V, D = emb.shape
    B, L = center_words.shape
    sc = _build_sc_kernel(V, D, B, L)
    partials = sc(emb, center_words.reshape(-1), center_words_len, mwe_words)
    return jnp.sum(partials) / jnp.float32(B * D)


# speedup vs baseline: 2.9819x; 1.2168x over previous
import functools

import jax
import jax.numpy as jnp
from jax import lax
from jax.experimental import pallas as pl
from jax.experimental.pallas import tpu as pltpu
from jax.experimental.pallas import tpu_sc as plsc


def _build_sc_kernel(V, D, B, L):
    info = plsc.get_sparse_core_info()
    NC, NS, LN = info.num_cores, info.num_subcores, info.num_lanes
    NW = NC * NS
    assert B % NW == 0 and D % LN == 0
    b_per_w = B // NW
    CB = 16
    n_chunks = b_per_w // CB
    rows_per_chunk = CB * L
    n_vregs = D // LN

    mesh = plsc.VectorSubcoreMesh(core_axis_name="c", subcore_axis_name="s")

    @functools.partial(
        pl.kernel,
        mesh=mesh,
        out_type=jax.ShapeDtypeStruct((NW, LN), jnp.float32),
        scratch_types=[
            pltpu.VMEM((b_per_w * L,), jnp.int32),
            pltpu.VMEM((b_per_w + LN,), jnp.int32),
            pltpu.VMEM((b_per_w,), jnp.int32),
            pltpu.VMEM((rows_per_chunk, D), jnp.float32),
            pltpu.VMEM((rows_per_chunk, D), jnp.float32),
            pltpu.VMEM((CB, D), jnp.float32),
            pltpu.VMEM((CB, D), jnp.float32),
            pltpu.VMEM((LN,), jnp.float32),
            pltpu.SemaphoreType.DMA,
            pltpu.SemaphoreType.DMA,
        ],
    )
    def sc_kernel(emb_hbm, cw_hbm, len_hbm, mwe_hbm, out_hbm,
                  idx_v, len_v, midx_v, rows_a, rows_b, tgt_a, tgt_b,
                  out_v, sem_a, sem_b):
        wid = lax.axis_index("s") * NC + lax.axis_index("c")
        base_b = wid * b_per_w

        pltpu.sync_copy(cw_hbm.at[pl.ds(base_b * L, b_per_w * L)], idx_v)
        pltpu.sync_copy(len_hbm.at[pl.ds(base_b, b_per_w)],
                        len_v.at[pl.ds(0, b_per_w)])
        pltpu.sync_copy(mwe_hbm.at[pl.ds(base_b, b_per_w)], midx_v)

        zeros = jnp.zeros((LN,), jnp.float32)
        sq = (zeros,) * n_vregs

        bufs = ((rows_a, tgt_a, sem_a), (rows_b, tgt_b, sem_b))

        def fire(cb):
            rows_v, tgt_v, sem = bufs[cb % 2]
            off = cb * rows_per_chunk
            cps = []
            r0 = 0
            while r0 < rows_per_chunk:
                n = min(128, rows_per_chunk - r0)
                cps.append(pltpu.async_copy(
                    emb_hbm.at[idx_v.at[pl.ds(off + r0, n)]],
                    rows_v.at[pl.ds(r0, n)], sem))
                r0 += n
            cps.append(pltpu.async_copy(
                emb_hbm.at[midx_v.at[pl.ds(cb * CB, CB)]], tgt_v, sem))
            return cps

        pending = fire(0)
        for cb in range(n_chunks):
            rows_v, tgt_v, _ = bufs[cb % 2]
            cur = pending
            if cb + 1 < n_chunks:
                pending = fire(cb + 1)
            for cp in cur:
                cp.wait()

            def b_body(b, sq, _cb=cb, rows_v=rows_v, tgt_v=tgt_v):
                ln = len_v[pl.ds(_cb * CB + b, LN)][0]
                inv = jnp.full((LN,), 1.0, jnp.float32) / jnp.full(
                    (LN,), ln).astype(jnp.float32)

                def l_body(l, accs, _b=b):
                    r = _b * L + l
                    return tuple(accs[j] + rows_v[r, pl.ds(j * LN, LN)]
                                 for j in range(n_vregs))

                accs = lax.fori_loop(0, ln, l_body, (zeros,) * n_vregs)
                out = []
                for j in range(n_vregs):
                    diff = accs[j] * inv - tgt_v[b, pl.ds(j * LN, LN)]
                    out.append(sq[j] + diff * diff)
                return tuple(out)

            sq = lax.fori_loop(0, CB, b_body, sq)

        total = sq[0]
        for j in range(1, n_vregs):
            total = total + sq[j]
        out_v[...] = total
        pltpu.sync_copy(out_v, out_hbm.at[wid])

    return sc_kernel


def kernel(emb, center_words, center_words_len, mwe_words):
    V, D = emb.shape
    B, L = center_words.shape
    sc = _build_sc_kernel(V, D, B, L)
    partials = sc(emb, center_words.reshape(-1), center_words_len, mwe_words)
    return jnp.sum(partials) / jnp.float32(B * D)


# speedup vs baseline: 3.0563x; 1.0249x over previous
import functools

import jax
import jax.numpy as jnp
from jax import lax
from jax.experimental import pallas as pl
from jax.experimental.pallas import tpu as pltpu
from jax.experimental.pallas import tpu_sc as plsc


def _build_sc_kernel(V, D, B, L):
    info = plsc.get_sparse_core_info()
    NC, NS, LN = info.num_cores, info.num_subcores, info.num_lanes
    NW = NC * NS
    assert B % NW == 0 and D % LN == 0
    b_per_w = B // NW
    CB = 16
    n_chunks = b_per_w // CB
    rows_per_chunk = CB * L
    n_vregs = D // LN

    mesh = plsc.VectorSubcoreMesh(core_axis_name="c", subcore_axis_name="s")

    @functools.partial(
        pl.kernel,
        mesh=mesh,
        out_type=jax.ShapeDtypeStruct((NW, LN), jnp.float32),
        scratch_types=[
            pltpu.VMEM((b_per_w * L,), jnp.int32),
            pltpu.VMEM((b_per_w + LN,), jnp.int32),
            pltpu.VMEM((b_per_w,), jnp.int32),
            pltpu.VMEM((rows_per_chunk, D), jnp.float32),
            pltpu.VMEM((rows_per_chunk, D), jnp.float32),
            pltpu.VMEM((CB, D), jnp.float32),
            pltpu.VMEM((CB, D), jnp.float32),
            pltpu.VMEM((LN,), jnp.float32),
            pltpu.SemaphoreType.DMA,
            pltpu.SemaphoreType.DMA,
        ],
    )
    def sc_kernel(emb_hbm, cw_hbm, len_hbm, mwe_hbm, out_hbm,
                  idx_v, len_v, midx_v, rows_a, rows_b, tgt_a, tgt_b,
                  out_v, sem_a, sem_b):
        wid = lax.axis_index("s") * NC + lax.axis_index("c")
        base_b = wid * b_per_w

        pltpu.sync_copy(cw_hbm.at[pl.ds(base_b * L, b_per_w * L)], idx_v)
        pltpu.sync_copy(len_hbm.at[pl.ds(base_b, b_per_w)],
                        len_v.at[pl.ds(0, b_per_w)])
        pltpu.sync_copy(mwe_hbm.at[pl.ds(base_b, b_per_w)], midx_v)

        zeros = jnp.zeros((LN,), jnp.float32)
        sq = (zeros,) * n_vregs

        bufs = ((rows_a, tgt_a, sem_a), (rows_b, tgt_b, sem_b))

        def fire(cb, par):
            rows_v, tgt_v, sem = bufs[par]
            off = pl.multiple_of(cb * rows_per_chunk, rows_per_chunk)
            for r0 in range(0, rows_per_chunk, 128):
                n = min(128, rows_per_chunk - r0)
                pltpu.async_copy(
                    emb_hbm.at[idx_v.at[pl.ds(off + r0, n)]],
                    rows_v.at[pl.ds(r0, n)], sem)
            pltpu.async_copy(
                emb_hbm.at[midx_v.at[pl.ds(pl.multiple_of(cb * CB, CB), CB)]],
                tgt_v, sem)

        def drain(par):
            rows_v, tgt_v, sem = bufs[par]
            pltpu.make_async_copy(
                emb_hbm.at[idx_v.at[pl.ds(0, rows_per_chunk)]],
                rows_v, sem).wait()
            pltpu.make_async_copy(
                emb_hbm.at[midx_v.at[pl.ds(0, CB)]], tgt_v, sem).wait()

        def compute(cb, par, sq):
            rows_v, tgt_v, _ = bufs[par]

            def b_body(b, sq, _cb=cb):
                ln = len_v[pl.ds(_cb * CB + b, LN)][0]
                inv = jnp.full((LN,), 1.0, jnp.float32) / jnp.full(
                    (LN,), ln).astype(jnp.float32)

                def l_body(l, accs, _b=b):
                    r = _b * L + l
                    return tuple(accs[j] + rows_v[r, pl.ds(j * LN, LN)]
                                 for j in range(n_vregs))

                accs = lax.fori_loop(0, ln, l_body, (zeros,) * n_vregs)
                out = []
                for j in range(n_vregs):
                    diff = accs[j] * inv - tgt_v[b, pl.ds(j * LN, LN)]
                    out.append(sq[j] + diff * diff)
                return tuple(out)

            return lax.fori_loop(0, CB, b_body, sq)

        fire(0, 0)
        fire(1, 1)
        n_pairs = n_chunks // 2

        def pair_body(k, sq):
            c = 2 * k
            drain(0)
            sq = compute(c, 0, sq)

            @pl.when(k < n_pairs - 1)
            def _():
                fire(c + 2, 0)

            drain(1)
            sq = compute(c + 1, 1, sq)

            @pl.when(k < n_pairs - 1)
            def _():
                fire(c + 3, 1)

            return sq

        sq = lax.fori_loop(0, n_pairs, pair_body, sq)

        total = sq[0]
        for j in range(1, n_vregs):
            total = total + sq[j]
        out_v[...] = total
        pltpu.sync_copy(out_v, out_hbm.at[wid])

    return sc_kernel


def kernel(emb, center_words, center_words_len, mwe_words):
    V, D = emb.shape
    B, L = center_words.shape
    sc = _build_sc_kernel(V, D, B, L)
    partials = sc(emb, center_words.reshape(-1), center_words_len, mwe_words)
    return jnp.sum(partials) / jnp.float32(B * D)
